# Initial kernel scaffold; baseline (speedup 1.0000x reference)
import jax
import jax.numpy as jnp
from jax.experimental import pallas as pl


def kernel(x, weight):
    raise NotImplementedError("write your pallas kernel here")


# speedup vs baseline: 1.5742x; 1.5742x over previous
import functools

import jax
import jax.numpy as jnp
from jax import lax
from jax.experimental import pallas as pl
from jax.experimental.pallas import tpu as pltpu
from jax.experimental.pallas import tpu_sc as plsc

NUM_EMB = 1000000
DIM = 64
B_TOTAL = 16384 * 50

_info = plsc.get_sparse_core_info()
NC, NS = _info.num_cores, _info.num_subcores
NW = NC * NS
BPW = B_TOTAL // NW
CHUNK = 128
NCHUNK = BPW // CHUNK

_mesh = plsc.VectorSubcoreMesh(core_axis_name="c", subcore_axis_name="s")


@functools.partial(
    pl.kernel,
    mesh=_mesh,
    out_type=jax.ShapeDtypeStruct((B_TOTAL, DIM), jnp.float32),
    scratch_types=[
        pltpu.VMEM((CHUNK,), jnp.int32),
        pltpu.VMEM((CHUNK, DIM), jnp.float32),
        pltpu.SemaphoreType.DMA,
    ],
    compiler_params=pltpu.CompilerParams(use_tc_tiling_on_sc=False),
)
def _emb_lookup(table_hbm, idx_hbm, out_hbm, idx_v, rows_v, sem):
    wid = lax.axis_index("s") * NC + lax.axis_index("c")
    base = wid * BPW

    def chunk_body(g, carry):
        off = base + g * CHUNK
        pltpu.sync_copy(idx_hbm.at[pl.ds(off, CHUNK)], idx_v)
        pltpu.async_copy(table_hbm.at[idx_v], rows_v, sem).wait()
        pltpu.sync_copy(rows_v, out_hbm.at[pl.ds(off, CHUNK)])
        return carry

    lax.fori_loop(0, NCHUNK, chunk_body, 0)


def kernel(x, weight):
    x_flat = x.reshape(-1).astype(jnp.int32)
    out = _emb_lookup(weight, x_flat)
    return out.reshape(x.shape + (DIM,))


# speedup vs baseline: 1.8744x; 1.1907x over previous
import functools

import jax
import jax.numpy as jnp
from jax import lax
from jax.experimental import pallas as pl
from jax.experimental.pallas import tpu as pltpu
from jax.experimental.pallas import tpu_sc as plsc

NUM_EMB = 1000000
DIM = 64
B_TOTAL = 16384 * 50

_info = plsc.get_sparse_core_info()
NC, NS = _info.num_cores, _info.num_subcores
NW = NC * NS
BPW = B_TOTAL // NW
CHUNK = 128
NCHUNK = BPW // CHUNK
NBUF = 8
NGROUP = NCHUNK // NBUF

_mesh = plsc.VectorSubcoreMesh(core_axis_name="c", subcore_axis_name="s")


@functools.partial(
    pl.kernel,
    mesh=_mesh,
    out_type=jax.ShapeDtypeStruct((B_TOTAL, DIM), jnp.float32),
    scratch_types=[
        pltpu.VMEM((NCHUNK, CHUNK), jnp.int32),
        pltpu.VMEM((NBUF, CHUNK, DIM), jnp.float32),
        pltpu.SemaphoreType.DMA((NBUF,)),
        pltpu.SemaphoreType.DMA((NBUF,)),
    ],
    compiler_params=pltpu.CompilerParams(use_tc_tiling_on_sc=False),
)
def _emb_lookup(table_hbm, idx_hbm, out_hbm, idx_all, rows, gsem, ssem):
    wid = lax.axis_index("s") * NC + lax.axis_index("c")
    base_chunk = wid * NCHUNK
    base = wid * BPW

    pltpu.sync_copy(idx_hbm.at[pl.ds(base_chunk, NCHUNK)], idx_all)

    def gather(g, b):
        pltpu.async_copy(table_hbm.at[idx_all.at[g]], rows.at[b], gsem.at[b])

    def store(g, b):
        pltpu.async_copy(rows.at[b], out_hbm.at[pl.ds(base + g * CHUNK, CHUNK)],
                         ssem.at[b])

    for b in range(NBUF):
        gather(b, b)

    def group_body(t, carry):
        for b in range(NBUF):
            g_done = (t - 1) * NBUF + b
            pltpu.make_async_copy(table_hbm.at[idx_all.at[g_done]], rows.at[b],
                                  gsem.at[b]).wait()
            store(g_done, b)
            pltpu.make_async_copy(rows.at[b],
                                  out_hbm.at[pl.ds(base + g_done * CHUNK, CHUNK)],
                                  ssem.at[b]).wait()
            gather(t * NBUF + b, b)
        return carry

    lax.fori_loop(1, NGROUP, group_body, 0)

    for b in range(NBUF):
        g_done = (NGROUP - 1) * NBUF + b
        pltpu.make_async_copy(table_hbm.at[idx_all.at[g_done]], rows.at[b],
                              gsem.at[b]).wait()
        store(g_done, b)
        pltpu.make_async_copy(rows.at[b],
                              out_hbm.at[pl.ds(base + g_done * CHUNK, CHUNK)],
                              ssem.at[b]).wait()


def kernel(x, weight):
    x_flat = x.reshape(NW * NCHUNK, CHUNK).astype(jnp.int32)
    out = _emb_lookup(weight, x_flat)
    return out.reshape(x.shape + (DIM,))
